# baseline (speedup 1.0000x reference)
import functools

import jax
import jax.numpy as jnp
from jax.experimental import pallas as pl


def _softargmax_block(scores_ref, out_ref, *, D):
    blk = scores_ref[0]
    _, hblk, W = blk.shape
    x = blk.reshape(D, hblk * W)
    m = jnp.max(x, axis=0, keepdims=True)
    log2e = jnp.float32(1.4426950408889634)
    e = jnp.exp2(x * log2e - m * log2e)
    c = (D - 1) * 0.5
    i = jax.lax.broadcasted_iota(jnp.int32, (2, D), 0)
    j = jax.lax.broadcasted_iota(jnp.int32, (2, D), 1)
    w = jnp.where(i == 0, j.astype(jnp.float32) - c, 1.0)
    r = jax.lax.dot_general(
        w, e, (((1,), (0,)), ((), ())), preferred_element_type=jnp.float32
    )
    out_ref[0, 0] = (r[0] / r[1] + c).reshape(hblk, W)


def kernel(scores):
    B, D, H, W = scores.shape
    hblk = 128
    grid = (B, H // hblk)
    out = pl.pallas_call(
        functools.partial(_softargmax_block, D=D),
        grid=grid,
        in_specs=[
            pl.BlockSpec((1, D, hblk, W), lambda b, h: (b, 0, h, 0)),
        ],
        out_specs=pl.BlockSpec((1, 1, hblk, W), lambda b, h: (b, 0, h, 0)),
        out_shape=jax.ShapeDtypeStruct((B, 1, H, W), scores.dtype),
    )(scores)
    return out


# speedup vs baseline: 1.0937x; 1.0937x over previous
import functools

import jax
import jax.numpy as jnp
from jax.experimental import pallas as pl


def _softargmax_one(sref, D):
    blk = sref[0]
    _, hblk, W = blk.shape
    x = blk.reshape(D, hblk * W)
    m = jnp.max(x, axis=0, keepdims=True)
    log2e = jnp.float32(1.4426950408889634)
    e = jnp.exp2(x * log2e - m * log2e)
    c = (D - 1) * 0.5
    i = jax.lax.broadcasted_iota(jnp.int32, (2, D), 0)
    j = jax.lax.broadcasted_iota(jnp.int32, (2, D), 1)
    w = jnp.where(i == 0, j.astype(jnp.float32) - c, 1.0)
    r = jax.lax.dot_general(
        w, e, (((1,), (0,)), ((), ())), preferred_element_type=jnp.float32
    )
    return (r[0] / r[1] + c).reshape(hblk, W)


def _softargmax_block(s0_ref, s1_ref, out_ref, *, D, hblk):
    out_ref[0, 0, :hblk] = _softargmax_one(s0_ref, D)
    out_ref[0, 0, hblk:] = _softargmax_one(s1_ref, D)


def kernel(scores):
    B, D, H, W = scores.shape
    hblk = 32
    grid = (B, H // (2 * hblk))
    out = pl.pallas_call(
        functools.partial(_softargmax_block, D=D, hblk=hblk),
        grid=grid,
        in_specs=[
            pl.BlockSpec((1, D, hblk, W), lambda b, h: (b, 0, 2 * h, 0)),
            pl.BlockSpec((1, D, hblk, W), lambda b, h: (b, 0, 2 * h + 1, 0)),
        ],
        out_specs=pl.BlockSpec((1, 1, 2 * hblk, W), lambda b, h: (b, 0, h, 0)),
        out_shape=jax.ShapeDtypeStruct((B, 1, H, W), scores.dtype),
    )(scores, scores)
    return out


# speedup vs baseline: 1.2334x; 1.1277x over previous
import functools

import jax
import jax.numpy as jnp
from jax.experimental import pallas as pl


def _softargmax_one(sref, D):
    blk = sref[0]
    _, hblk, W = blk.shape
    x = blk.reshape(D, hblk * W)
    m = jnp.max(x, axis=0, keepdims=True)
    log2e = jnp.float32(1.4426950408889634)
    e = jnp.exp2(x * log2e - m * log2e)
    c = (D - 1) * 0.5
    i = jax.lax.broadcasted_iota(jnp.int32, (2, D), 0)
    j = jax.lax.broadcasted_iota(jnp.int32, (2, D), 1)
    w = jnp.where(i == 0, j.astype(jnp.float32) - c, 1.0)
    r = jax.lax.dot_general(
        w, e, (((1,), (0,)), ((), ())), preferred_element_type=jnp.float32
    )
    return (r[0] / r[1] + c).reshape(hblk, W)


def _softargmax_block(s0_ref, s1_ref, s2_ref, s3_ref, out_ref, *, D, hblk):
    for k, sref in enumerate((s0_ref, s1_ref, s2_ref, s3_ref)):
        out_ref[0, 0, k * hblk : (k + 1) * hblk] = _softargmax_one(sref, D)


def kernel(scores):
    B, D, H, W = scores.shape
    S = 4
    hblk = 16
    grid = (B, H // (S * hblk))
    in_specs = [
        pl.BlockSpec((1, D, hblk, W), functools.partial(lambda k, b, h: (b, 0, S * h + k, 0), k))
        for k in range(S)
    ]
    out = pl.pallas_call(
        functools.partial(_softargmax_block, D=D, hblk=hblk),
        grid=grid,
        in_specs=in_specs,
        out_specs=pl.BlockSpec((1, 1, S * hblk, W), lambda b, h: (b, 0, h, 0)),
        out_shape=jax.ShapeDtypeStruct((B, 1, H, W), scores.dtype),
    )(scores, scores, scores, scores)
    return out
